# baseline (speedup 1.0000x reference)
import functools

import jax
import jax.numpy as jnp
from jax import lax
from jax.experimental import pallas as pl
from jax.experimental.pallas import tpu as pltpu
from jax.experimental.pallas import tpu_sc as plsc

_LANES = 16
_IDX_CHUNK = 128


@jax.jit
def kernel(node, time, embedding):
    B = node.shape[0]
    D = embedding.shape[1]
    info = plsc.get_sparse_core_info()
    nw = info.num_cores * info.num_subcores
    b_per_w = B // nw
    n_chunks = b_per_w // _IDX_CHUNK

    mesh = plsc.VectorSubcoreMesh(core_axis_name="c", subcore_axis_name="s")

    @functools.partial(
        pl.kernel,
        mesh=mesh,
        compiler_params=pltpu.CompilerParams(
            needs_layout_passes=False, use_tc_tiling_on_sc=False),
        out_type=jax.ShapeDtypeStruct((nw, b_per_w), jnp.float32),
        scratch_types=[
            pltpu.VMEM((n_chunks, _IDX_CHUNK), jnp.int32),
            pltpu.VMEM((n_chunks, _IDX_CHUNK), jnp.int32),
            pltpu.VMEM((b_per_w, D), jnp.float32),
            pltpu.VMEM((b_per_w, D), jnp.float32),
            pltpu.VMEM((b_per_w,), jnp.float32),
            pltpu.VMEM((_LANES * _LANES,), jnp.float32),
            pltpu.SemaphoreType.DMA,
        ],
    )
    def sc_kernel(node_hbm, time_hbm, emb_hbm, out_hbm,
                  idx_n, idx_t, rows_n, rows_t, out_v, pbuf, sem):
        c = lax.axis_index("c")
        s = lax.axis_index("s")
        wid = s * info.num_cores + c

        pltpu.sync_copy(node_hbm.at[wid], idx_n)
        pltpu.sync_copy(time_hbm.at[wid], idx_t)

        copies = []
        for j in range(n_chunks):
            dst = pl.ds(j * _IDX_CHUNK, _IDX_CHUNK)
            copies.append(
                pltpu.async_copy(emb_hbm.at[idx_n.at[j]], rows_n.at[dst], sem))
            copies.append(
                pltpu.async_copy(emb_hbm.at[idx_t.at[j]], rows_t.at[dst], sem))
        for cp in copies:
            cp.wait()

        sidx = lax.iota(jnp.int32, _LANES) * _LANES

        def group_body(g, carry):
            row0 = g * _LANES
            for r in range(_LANES):
                b = row0 + r
                pv = jnp.zeros((_LANES,), jnp.float32)
                for h in range(D // _LANES):
                    cs = pl.ds(h * _LANES, _LANES)
                    pv = pv + rows_n[b, cs] * rows_t[b, cs]
                plsc.store_scatter(pbuf, [sidx + r], pv)
            acc = jnp.zeros((_LANES,), jnp.float32)
            for j in range(_LANES):
                acc = acc + pbuf[pl.ds(j * _LANES, _LANES)]
            out_v[pl.ds(row0, _LANES)] = acc
            return carry

        lax.fori_loop(0, b_per_w // _LANES, group_body, 0)

        pltpu.sync_copy(out_v, out_hbm.at[wid])

    node_r = node.astype(jnp.int32).reshape(nw, n_chunks, _IDX_CHUNK)
    time_r = time.astype(jnp.int32).reshape(nw, n_chunks, _IDX_CHUNK)
    out = sc_kernel(node_r, time_r, embedding)
    return out.reshape(B)
